# Initial kernel scaffold; baseline (speedup 1.0000x reference)
import functools

import jax
import jax.numpy as jnp
from jax import lax
from jax.experimental import pallas as pl
from jax.experimental.pallas import tpu as pltpu
from jax.experimental.pallas import tpu_sc as plsc

IN_CH = 256
OUT_CH = 256
N_NODES = 10000
N_EDGES = 160000

CA = 272
NUM_SC = 2
NUM_TILES = 16
NHALF = N_NODES // NUM_SC
ROWS_SH = 5008
ZROWS = ROWS_SH // NUM_TILES
DUMMY = NHALF
CHUNK = 128
E_PAD = 163840
N_CHUNKS = E_PAD // (NUM_TILES * CHUNK)

_mesh = plsc.VectorSubcoreMesh(core_axis_name="c", subcore_axis_name="s")


@functools.partial(
    pl.kernel,
    out_type=jax.ShapeDtypeStruct((NUM_SC, ROWS_SH, CA), jnp.float32),
    mesh=_mesh,
    scratch_types=[
        pltpu.VMEM((N_CHUNKS, CHUNK), jnp.int32),
        pltpu.VMEM((N_CHUNKS, CHUNK), jnp.int32),
        pltpu.VMEM((CHUNK, CA), jnp.float32),
        pltpu.VMEM_SHARED((ROWS_SH, CA), jnp.float32),
        pltpu.SemaphoreType.DMA,
    ],
)
def _sc_aggregate(xa_hbm, src_hbm, dst_hbm, zeros_hbm, out_hbm,
                  src_v, lidx_v, rows_v, agg_sh, sem):
    cid = lax.axis_index("c")
    sid = lax.axis_index("s")
    base = cid * NHALF

    pltpu.sync_copy(src_hbm.at[sid], src_v)
    pltpu.sync_copy(dst_hbm.at[sid], lidx_v)
    pltpu.sync_copy(zeros_hbm, agg_sh.at[pl.ds(sid * ZROWS, ZROWS)])

    def idx_body(j, _):
        for k in range(CHUNK // 16):
            d = lidx_v[j, pl.ds(k * 16, 16)]
            ok = (d >= base) & (d < base + NHALF)
            lidx_v[j, pl.ds(k * 16, 16)] = jnp.where(ok, d - base, DUMMY)
        return 0

    lax.fori_loop(0, N_CHUNKS, idx_body, 0)
    plsc.subcore_barrier()

    def chunk_body(j, _):
        pltpu.async_copy(xa_hbm.at[src_v.at[j]], rows_v, sem).wait()
        pltpu.sync_copy(rows_v, agg_sh.at[lidx_v.at[j]], add=True)
        return 0

    lax.fori_loop(0, N_CHUNKS, chunk_body, 0)
    plsc.subcore_barrier()

    pltpu.sync_copy(agg_sh.at[pl.ds(sid * ZROWS, ZROWS)],
                    out_hbm.at[cid, pl.ds(sid * ZROWS, ZROWS)])


_BR = 400


def _tc_body(x_ref, s_ref, c_ref, w1_ref, w2_ref, b1_ref, b2_ref, o_ref):
    c = c_ref[...]
    inv = 1.0 / jnp.maximum(c, 1.0)
    mean = s_ref[...] * inv
    acc = jnp.dot(x_ref[...], w1_ref[...], preferred_element_type=jnp.float32)
    acc = acc + jnp.dot(mean, w2_ref[...], preferred_element_type=jnp.float32)
    acc = acc + b1_ref[...]
    acc = acc + jnp.where(c > 0.0, 1.0, 0.0) * b2_ref[...]
    o_ref[...] = acc


def _tc_combine(x, sums, cnt, W1, W2, b1, b2):
    return pl.pallas_call(
        _tc_body,
        grid=(N_NODES // _BR,),
        in_specs=[
            pl.BlockSpec((_BR, IN_CH), lambda i: (i, 0)),
            pl.BlockSpec((_BR, IN_CH), lambda i: (i, 0)),
            pl.BlockSpec((_BR, 1), lambda i: (i, 0)),
            pl.BlockSpec((IN_CH, OUT_CH), lambda i: (0, 0)),
            pl.BlockSpec((IN_CH, OUT_CH), lambda i: (0, 0)),
            pl.BlockSpec((1, OUT_CH), lambda i: (0, 0)),
            pl.BlockSpec((1, OUT_CH), lambda i: (0, 0)),
        ],
        out_specs=pl.BlockSpec((_BR, OUT_CH), lambda i: (i, 0)),
        out_shape=jax.ShapeDtypeStruct((N_NODES, OUT_CH), jnp.float32),
    )(x, sums, cnt, W1, W2, b1, b2)


def kernel(x, edge_index, W1, b1, W2, b2):
    src = edge_index[0].astype(jnp.int32)
    dst = edge_index[1].astype(jnp.int32)
    src_p = jnp.pad(src, (0, E_PAD - N_EDGES)).reshape(NUM_TILES, N_CHUNKS, CHUNK)
    dst_p = jnp.pad(dst, (0, E_PAD - N_EDGES),
                    constant_values=N_NODES).reshape(NUM_TILES, N_CHUNKS, CHUNK)
    xa = jnp.concatenate(
        [x, jnp.ones((N_NODES, 1), jnp.float32),
         jnp.zeros((N_NODES, CA - IN_CH - 1), jnp.float32)], axis=1)
    zeros = jnp.zeros((ZROWS, CA), jnp.float32)

    agg = _sc_aggregate(xa, src_p, dst_p, zeros)
    sums = agg[:, :NHALF, :IN_CH].reshape(N_NODES, IN_CH)
    cnt = agg[:, :NHALF, IN_CH].reshape(N_NODES, 1)

    return _tc_combine(x, sums, cnt, W1, W2, b1, b2)


# speedup vs baseline: 2.1977x; 2.1977x over previous
import functools

import jax
import jax.numpy as jnp
from jax import lax
from jax.experimental import pallas as pl
from jax.experimental.pallas import tpu as pltpu
from jax.experimental.pallas import tpu_sc as plsc

IN_CH = 256
OUT_CH = 256
N_NODES = 10000
N_EDGES = 160000

CA = 272
NUM_SC = 2
NUM_TILES = 16
NHALF = N_NODES // NUM_SC
ROWS_SH = 5120
ZROWS = ROWS_SH // NUM_TILES
DUMMY = NHALF
CHUNK = 128
E_PAD = 163840
N_CHUNKS = E_PAD // (NUM_TILES * CHUNK)
SB = 8
N_SUPER = N_CHUNKS // SB

_mesh = plsc.VectorSubcoreMesh(core_axis_name="c", subcore_axis_name="s")


@functools.partial(
    pl.kernel,
    out_type=jax.ShapeDtypeStruct((NUM_SC, ROWS_SH, CA), jnp.float32),
    mesh=_mesh,
    scratch_types=[
        pltpu.VMEM((SB, CHUNK), jnp.int32),
        pltpu.VMEM((SB, CHUNK), jnp.int32),
        pltpu.VMEM((CHUNK, CA), jnp.float32),
        pltpu.VMEM_SHARED((ROWS_SH, CA), jnp.float32),
        pltpu.SemaphoreType.DMA,
    ],
    compiler_params=pltpu.CompilerParams(use_tc_tiling_on_sc=False),
)
def _sc_aggregate(xa_hbm, src_hbm, dst_hbm, zeros_hbm, out_hbm,
                  src_v, lidx_v, rows_v, agg_sh, sem):
    cid = lax.axis_index("c")
    sid = lax.axis_index("s")
    base = cid * NHALF

    pltpu.sync_copy(zeros_hbm, agg_sh.at[pl.ds(sid * ZROWS, ZROWS)])
    plsc.subcore_barrier()

    def super_body(s, _):
        pltpu.sync_copy(src_hbm.at[sid, pl.ds(s * SB, SB)], src_v)
        pltpu.sync_copy(dst_hbm.at[sid, pl.ds(s * SB, SB)], lidx_v)
        for j in range(SB):
            for k in range(CHUNK // 16):
                d = lidx_v[j, pl.ds(k * 16, 16)]
                ok = (d >= base) & (d < base + NHALF)
                lidx_v[j, pl.ds(k * 16, 16)] = jnp.where(ok, d - base, DUMMY)
        for j in range(SB):
            pltpu.async_copy(xa_hbm.at[src_v.at[j]], rows_v, sem).wait()
            pltpu.sync_copy(rows_v, agg_sh.at[lidx_v.at[j]], add=True)
        return 0

    lax.fori_loop(0, N_SUPER, super_body, 0)
    plsc.subcore_barrier()

    pltpu.sync_copy(agg_sh.at[pl.ds(sid * ZROWS, ZROWS)],
                    out_hbm.at[cid, pl.ds(sid * ZROWS, ZROWS)])


_BR = 400


def _tc_body(x_ref, s_ref, c_ref, w1_ref, w2_ref, b1_ref, b2_ref, o_ref):
    c = c_ref[...]
    inv = 1.0 / jnp.maximum(c, 1.0)
    mean = s_ref[...] * inv
    acc = jnp.dot(x_ref[...], w1_ref[...], preferred_element_type=jnp.float32)
    acc = acc + jnp.dot(mean, w2_ref[...], preferred_element_type=jnp.float32)
    acc = acc + b1_ref[...]
    acc = acc + jnp.where(c > 0.0, 1.0, 0.0) * b2_ref[...]
    o_ref[...] = acc


def _tc_combine(x, sums, cnt, W1, W2, b1, b2):
    return pl.pallas_call(
        _tc_body,
        grid=(N_NODES // _BR,),
        in_specs=[
            pl.BlockSpec((_BR, IN_CH), lambda i: (i, 0)),
            pl.BlockSpec((_BR, IN_CH), lambda i: (i, 0)),
            pl.BlockSpec((_BR, 1), lambda i: (i, 0)),
            pl.BlockSpec((IN_CH, OUT_CH), lambda i: (0, 0)),
            pl.BlockSpec((IN_CH, OUT_CH), lambda i: (0, 0)),
            pl.BlockSpec((1, OUT_CH), lambda i: (0, 0)),
            pl.BlockSpec((1, OUT_CH), lambda i: (0, 0)),
        ],
        out_specs=pl.BlockSpec((_BR, OUT_CH), lambda i: (i, 0)),
        out_shape=jax.ShapeDtypeStruct((N_NODES, OUT_CH), jnp.float32),
    )(x, sums, cnt, W1, W2, b1, b2)


def kernel(x, edge_index, W1, b1, W2, b2):
    src = edge_index[0].astype(jnp.int32)
    dst = edge_index[1].astype(jnp.int32)
    src_p = jnp.pad(src, (0, E_PAD - N_EDGES)).reshape(NUM_TILES, N_CHUNKS, CHUNK)
    dst_p = jnp.pad(dst, (0, E_PAD - N_EDGES),
                    constant_values=N_NODES).reshape(NUM_TILES, N_CHUNKS, CHUNK)
    xa = jnp.concatenate(
        [x, jnp.ones((N_NODES, 1), jnp.float32),
         jnp.zeros((N_NODES, CA - IN_CH - 1), jnp.float32)], axis=1)
    zeros = jnp.zeros((ZROWS, CA), jnp.float32)

    agg = _sc_aggregate(xa, src_p, dst_p, zeros)
    sums = agg[:, :NHALF, :IN_CH].reshape(N_NODES, IN_CH)
    cnt = agg[:, :NHALF, IN_CH].reshape(N_NODES, 1)

    return _tc_combine(x, sums, cnt, W1, W2,
                       b1.reshape(1, OUT_CH), b2.reshape(1, OUT_CH))


# speedup vs baseline: 2.4446x; 1.1123x over previous
import functools

import jax
import jax.numpy as jnp
from jax import lax
from jax.experimental import pallas as pl
from jax.experimental.pallas import tpu as pltpu
from jax.experimental.pallas import tpu_sc as plsc

IN_CH = 256
OUT_CH = 256
N_NODES = 10000
N_EDGES = 160000

CA = 272
NUM_SC = 2
NUM_TILES = 16
NHALF = N_NODES // NUM_SC
ROWS_SH = 5120
ZROWS = ROWS_SH // NUM_TILES
DUMMY = NHALF
CHUNK = 64
E_PAD = 163840
N_CHUNKS = E_PAD // (NUM_TILES * CHUNK)
SB = 8
N_SUPER = N_CHUNKS // SB

_mesh = plsc.VectorSubcoreMesh(core_axis_name="c", subcore_axis_name="s")


@functools.partial(
    pl.kernel,
    out_type=jax.ShapeDtypeStruct((NUM_SC, ROWS_SH, CA), jnp.float32),
    mesh=_mesh,
    scratch_types=[
        pltpu.VMEM((SB, CHUNK), jnp.int32),
        pltpu.VMEM((SB, CHUNK), jnp.int32),
        pltpu.VMEM((CHUNK, CA), jnp.float32),
        pltpu.VMEM((CHUNK, CA), jnp.float32),
        pltpu.VMEM_SHARED((ROWS_SH, CA), jnp.float32),
        pltpu.SemaphoreType.DMA,
        pltpu.SemaphoreType.DMA,
        pltpu.SemaphoreType.DMA,
        pltpu.SemaphoreType.DMA,
    ],
    compiler_params=pltpu.CompilerParams(use_tc_tiling_on_sc=False),
)
def _sc_aggregate(xa_hbm, src_hbm, dst_hbm, zeros_hbm, out_hbm,
                  src_v, lidx_v, rows0, rows1, agg_sh,
                  semg0, semg1, sems0, sems1):
    cid = lax.axis_index("c")
    sid = lax.axis_index("s")
    base = cid * NHALF
    rows = (rows0, rows1)
    semg = (semg0, semg1)
    sems = (sems0, sems1)

    pltpu.sync_copy(zeros_hbm, agg_sh.at[pl.ds(sid * ZROWS, ZROWS)])
    plsc.subcore_barrier()

    def super_body(s, _):
        pltpu.sync_copy(src_hbm.at[sid, pl.ds(s * SB, SB)], src_v)
        pltpu.sync_copy(dst_hbm.at[sid, pl.ds(s * SB, SB)], lidx_v)
        for j in range(SB):
            for k in range(CHUNK // 16):
                d = lidx_v[j, pl.ds(k * 16, 16)]
                ok = (d >= base) & (d < base + NHALF)
                lidx_v[j, pl.ds(k * 16, 16)] = jnp.where(ok, d - base, DUMMY)
        gd = {}
        sd = {}
        gd[0] = pltpu.async_copy(xa_hbm.at[src_v.at[0]], rows[0], semg[0])
        for j in range(SB):
            b = j & 1
            if j + 1 < SB:
                if j >= 1:
                    sd[j - 1].wait()
                gd[j + 1] = pltpu.async_copy(
                    xa_hbm.at[src_v.at[j + 1]], rows[1 - b], semg[1 - b])
            gd[j].wait()
            sd[j] = pltpu.async_copy(
                rows[b], agg_sh.at[lidx_v.at[j]], sems[b], add=True)
        sd[SB - 2].wait()
        sd[SB - 1].wait()
        return 0

    lax.fori_loop(0, N_SUPER, super_body, 0)
    plsc.subcore_barrier()

    pltpu.sync_copy(agg_sh.at[pl.ds(sid * ZROWS, ZROWS)],
                    out_hbm.at[cid, pl.ds(sid * ZROWS, ZROWS)])


_BR = 400


def _tc_body(x_ref, s_ref, c_ref, w1_ref, w2_ref, b1_ref, b2_ref, o_ref):
    c = c_ref[...]
    inv = 1.0 / jnp.maximum(c, 1.0)
    mean = s_ref[...] * inv
    acc = jnp.dot(x_ref[...], w1_ref[...], preferred_element_type=jnp.float32)
    acc = acc + jnp.dot(mean, w2_ref[...], preferred_element_type=jnp.float32)
    acc = acc + b1_ref[...]
    acc = acc + jnp.where(c > 0.0, 1.0, 0.0) * b2_ref[...]
    o_ref[...] = acc


def _tc_combine(x, sums, cnt, W1, W2, b1, b2):
    return pl.pallas_call(
        _tc_body,
        grid=(N_NODES // _BR,),
        in_specs=[
            pl.BlockSpec((_BR, IN_CH), lambda i: (i, 0)),
            pl.BlockSpec((_BR, IN_CH), lambda i: (i, 0)),
            pl.BlockSpec((_BR, 1), lambda i: (i, 0)),
            pl.BlockSpec((IN_CH, OUT_CH), lambda i: (0, 0)),
            pl.BlockSpec((IN_CH, OUT_CH), lambda i: (0, 0)),
            pl.BlockSpec((1, OUT_CH), lambda i: (0, 0)),
            pl.BlockSpec((1, OUT_CH), lambda i: (0, 0)),
        ],
        out_specs=pl.BlockSpec((_BR, OUT_CH), lambda i: (i, 0)),
        out_shape=jax.ShapeDtypeStruct((N_NODES, OUT_CH), jnp.float32),
    )(x, sums, cnt, W1, W2, b1, b2)


def kernel(x, edge_index, W1, b1, W2, b2):
    src = edge_index[0].astype(jnp.int32)
    dst = edge_index[1].astype(jnp.int32)
    src_p = jnp.pad(src, (0, E_PAD - N_EDGES)).reshape(NUM_TILES, N_CHUNKS, CHUNK)
    dst_p = jnp.pad(dst, (0, E_PAD - N_EDGES),
                    constant_values=N_NODES).reshape(NUM_TILES, N_CHUNKS, CHUNK)
    xa = jnp.concatenate(
        [x, jnp.ones((N_NODES, 1), jnp.float32),
         jnp.zeros((N_NODES, CA - IN_CH - 1), jnp.float32)], axis=1)
    zeros = jnp.zeros((ZROWS, CA), jnp.float32)

    agg = _sc_aggregate(xa, src_p, dst_p, zeros)
    sums = agg[:, :NHALF, :IN_CH].reshape(N_NODES, IN_CH)
    cnt = agg[:, :NHALF, IN_CH].reshape(N_NODES, 1)

    return _tc_combine(x, sums, cnt, W1, W2,
                       b1.reshape(1, OUT_CH), b2.reshape(1, OUT_CH))


# speedup vs baseline: 4.6866x; 1.9171x over previous
import functools

import jax
import jax.numpy as jnp
from jax import lax
from jax.experimental import pallas as pl
from jax.experimental.pallas import tpu as pltpu
from jax.experimental.pallas import tpu_sc as plsc

IN_CH = 256
OUT_CH = 256
N_NODES = 10000
N_EDGES = 160000

CA = 272
NUM_SC = 2
NUM_TILES = 16
NHALF = N_NODES // NUM_SC
ROWS_SH = 5120
ZROWS = ROWS_SH // NUM_TILES
SENT = -1
CHUNK = 64
E_PAD = 163840
N_CHUNKS = E_PAD // (NUM_TILES * CHUNK)
SB = 8
N_SUPER = N_CHUNKS // SB

_mesh = plsc.VectorSubcoreMesh(core_axis_name="c", subcore_axis_name="s")


@functools.partial(
    pl.kernel,
    out_type=jax.ShapeDtypeStruct((NUM_SC, ROWS_SH, CA), jnp.float32),
    mesh=_mesh,
    scratch_types=[
        pltpu.VMEM((SB, CHUNK), jnp.int32),
        pltpu.VMEM((SB, CHUNK), jnp.int32),
        pltpu.VMEM((CHUNK, CA), jnp.float32),
        pltpu.VMEM((CHUNK, CA), jnp.float32),
        pltpu.VMEM_SHARED((ROWS_SH, CA), jnp.float32),
        pltpu.SemaphoreType.DMA,
        pltpu.SemaphoreType.DMA,
        pltpu.SemaphoreType.DMA,
        pltpu.SemaphoreType.DMA,
    ],
    compiler_params=pltpu.CompilerParams(use_tc_tiling_on_sc=False),
)
def _sc_aggregate(xa_hbm, src_hbm, dst_hbm, zeros_hbm, out_hbm,
                  src_v, lidx_v, rows0, rows1, agg_sh,
                  semg0, semg1, sems0, sems1):
    cid = lax.axis_index("c")
    sid = lax.axis_index("s")
    base = cid * NHALF
    rows = (rows0, rows1)
    semg = (semg0, semg1)
    sems = (sems0, sems1)

    pltpu.sync_copy(zeros_hbm, agg_sh.at[pl.ds(sid * ZROWS, ZROWS)])
    plsc.subcore_barrier()

    def super_body(s, _):
        pltpu.sync_copy(src_hbm.at[sid, pl.ds(s * SB, SB)], src_v)
        pltpu.sync_copy(dst_hbm.at[sid, pl.ds(s * SB, SB)], lidx_v)
        for j in range(SB):
            for k in range(CHUNK // 16):
                d = lidx_v[j, pl.ds(k * 16, 16)]
                sidx = src_v[j, pl.ds(k * 16, 16)]
                ok = (d >= base) & (d < base + NHALF)
                lidx_v[j, pl.ds(k * 16, 16)] = jnp.where(ok, d - base, SENT)
                src_v[j, pl.ds(k * 16, 16)] = jnp.where(ok, sidx, SENT)
        gd = {}
        sd = {}

        def gather(j, buf, sem):
            return pltpu.async_copy(
                xa_hbm.at[plsc.Indices(src_v.at[j], ignored_value=SENT)],
                buf, sem)

        gd[0] = gather(0, rows[0], semg[0])
        for j in range(SB):
            b = j & 1
            if j + 1 < SB:
                if j >= 1:
                    sd[j - 1].wait()
                gd[j + 1] = gather(j + 1, rows[1 - b], semg[1 - b])
            gd[j].wait()
            sd[j] = pltpu.async_copy(
                rows[b],
                agg_sh.at[plsc.Indices(lidx_v.at[j], ignored_value=SENT)],
                sems[b], add=True)
        sd[SB - 2].wait()
        sd[SB - 1].wait()
        return 0

    lax.fori_loop(0, N_SUPER, super_body, 0)
    plsc.subcore_barrier()

    pltpu.sync_copy(agg_sh.at[pl.ds(sid * ZROWS, ZROWS)],
                    out_hbm.at[cid, pl.ds(sid * ZROWS, ZROWS)])


_BR = 400


def _tc_body(x_ref, s_ref, c_ref, w1_ref, w2_ref, b1_ref, b2_ref, o_ref):
    c = c_ref[...]
    inv = 1.0 / jnp.maximum(c, 1.0)
    mean = s_ref[...] * inv
    acc = jnp.dot(x_ref[...], w1_ref[...], preferred_element_type=jnp.float32)
    acc = acc + jnp.dot(mean, w2_ref[...], preferred_element_type=jnp.float32)
    acc = acc + b1_ref[...]
    acc = acc + jnp.where(c > 0.0, 1.0, 0.0) * b2_ref[...]
    o_ref[...] = acc


def _tc_combine(x, sums, cnt, W1, W2, b1, b2):
    return pl.pallas_call(
        _tc_body,
        grid=(N_NODES // _BR,),
        in_specs=[
            pl.BlockSpec((_BR, IN_CH), lambda i: (i, 0)),
            pl.BlockSpec((_BR, IN_CH), lambda i: (i, 0)),
            pl.BlockSpec((_BR, 1), lambda i: (i, 0)),
            pl.BlockSpec((IN_CH, OUT_CH), lambda i: (0, 0)),
            pl.BlockSpec((IN_CH, OUT_CH), lambda i: (0, 0)),
            pl.BlockSpec((1, OUT_CH), lambda i: (0, 0)),
            pl.BlockSpec((1, OUT_CH), lambda i: (0, 0)),
        ],
        out_specs=pl.BlockSpec((_BR, OUT_CH), lambda i: (i, 0)),
        out_shape=jax.ShapeDtypeStruct((N_NODES, OUT_CH), jnp.float32),
    )(x, sums, cnt, W1, W2, b1, b2)


def kernel(x, edge_index, W1, b1, W2, b2):
    src = edge_index[0].astype(jnp.int32)
    dst = edge_index[1].astype(jnp.int32)
    src_p = jnp.pad(src, (0, E_PAD - N_EDGES)).reshape(NUM_TILES, N_CHUNKS, CHUNK)
    dst_p = jnp.pad(dst, (0, E_PAD - N_EDGES),
                    constant_values=N_NODES).reshape(NUM_TILES, N_CHUNKS, CHUNK)
    xa = jnp.concatenate(
        [x, jnp.ones((N_NODES, 1), jnp.float32),
         jnp.zeros((N_NODES, CA - IN_CH - 1), jnp.float32)], axis=1)
    zeros = jnp.zeros((ZROWS, CA), jnp.float32)

    agg = _sc_aggregate(xa, src_p, dst_p, zeros)
    sums = agg[:, :NHALF, :IN_CH].reshape(N_NODES, IN_CH)
    cnt = agg[:, :NHALF, IN_CH].reshape(N_NODES, 1)

    return _tc_combine(x, sums, cnt, W1, W2,
                       b1.reshape(1, OUT_CH), b2.reshape(1, OUT_CH))
